# Initial kernel scaffold; baseline (speedup 1.0000x reference)
import jax
import jax.numpy as jnp
from jax.experimental import pallas as pl


def kernel(z, embedding_weight, proj_weight):
    raise NotImplementedError("write your pallas kernel here")


# speedup vs baseline: 1.3577x; 1.3577x over previous
import functools

import jax
import jax.numpy as jnp
from jax import lax
from jax.experimental import pallas as pl
from jax.experimental.pallas import tpu as pltpu
from jax.experimental.pallas import tpu_sc as plsc

B_TOTAL = 16384
D = 256
N_CB = 8192

M_TILE = 512
N_CHUNK = 2048

ST_TILE = 2048

_DIMNUMS_LAST = (((1,), (1,)), ((), ()))


def _codebook_body(e_ref, w_ref, cb_ref, cn_ref):
    cb = lax.dot_general(e_ref[...], w_ref[...], _DIMNUMS_LAST,
                         preferred_element_type=jnp.float32)
    cb_ref[...] = cb
    cn_ref[...] = jnp.sum(cb * cb, axis=1, keepdims=True)


def _compute_codebook(embedding_weight, proj_weight):
    return pl.pallas_call(
        _codebook_body,
        out_shape=(
            jax.ShapeDtypeStruct((N_CB, D), jnp.float32),
            jax.ShapeDtypeStruct((N_CB, 1), jnp.float32),
        ),
    )(embedding_weight, proj_weight)


def _argmin_body(z_ref, cb_ref, cn_ref, out_ref):
    z_t = z_ref[...]
    zn = jnp.sum(z_t * z_t, axis=1, keepdims=True)
    run_min = jnp.full((M_TILE, 1), jnp.inf, jnp.float32)
    run_idx = jnp.zeros((M_TILE, 1), jnp.int32)
    for c in range(N_CB // N_CHUNK):
        cb_c = cb_ref[pl.ds(c * N_CHUNK, N_CHUNK), :]
        s = lax.dot_general(z_t, cb_c, _DIMNUMS_LAST,
                            preferred_element_type=jnp.float32)
        d = (zn + cn_ref[:, pl.ds(c * N_CHUNK, N_CHUNK)]) - 2.0 * s
        cmin = jnp.min(d, axis=1, keepdims=True)
        io = lax.broadcasted_iota(jnp.int32, (M_TILE, N_CHUNK), 1) + (c * N_CHUNK)
        cidx = jnp.min(jnp.where(d == cmin, io, jnp.int32(2**30)),
                       axis=1, keepdims=True)
        upd = cmin < run_min
        run_min = jnp.where(upd, cmin, run_min)
        run_idx = jnp.where(upd, cidx, run_idx)
    out_ref[...] = run_idx[:, 0]


def _compute_indices(z_flat, codebook, cnorm_row):
    return pl.pallas_call(
        _argmin_body,
        grid=(B_TOTAL // M_TILE,),
        in_specs=[
            pl.BlockSpec((M_TILE, D), lambda i: (i, 0)),
            pl.BlockSpec((N_CB, D), lambda i: (0, 0)),
            pl.BlockSpec((1, N_CB), lambda i: (0, 0)),
        ],
        out_specs=pl.BlockSpec((M_TILE,), lambda i: (i,)),
        out_shape=jax.ShapeDtypeStruct((B_TOTAL,), jnp.int32),
    )(z_flat, codebook, cnorm_row)


_NUM_SC_CORES = 2
_NUM_SC_SUBCORES = 16
_NW = _NUM_SC_CORES * _NUM_SC_SUBCORES
_ROWS_PER_W = B_TOTAL // _NW
_GCHUNK = 128


def _gather_rows(codebook, indices):
    mesh = plsc.VectorSubcoreMesh(core_axis_name="c", subcore_axis_name="s")

    @functools.partial(
        pl.kernel, mesh=mesh,
        out_type=jax.ShapeDtypeStruct((B_TOTAL, D), jnp.float32),
        scratch_types=[
            pltpu.VMEM((_GCHUNK,), jnp.int32),
            pltpu.VMEM((_GCHUNK, D), jnp.float32),
            pltpu.SemaphoreType.DMA,
        ],
    )
    def gather_k(table_hbm, idx_hbm, out_hbm, idx_v, rows_v, sem):
        wid = lax.axis_index("s") * _NUM_SC_CORES + lax.axis_index("c")
        base = wid * _ROWS_PER_W
        for c in range(_ROWS_PER_W // _GCHUNK):
            off = base + c * _GCHUNK
            pltpu.sync_copy(idx_hbm.at[pl.ds(off, _GCHUNK)], idx_v)
            pltpu.async_copy(table_hbm.at[idx_v], rows_v, sem).wait()
            pltpu.sync_copy(rows_v, out_hbm.at[pl.ds(off, _GCHUNK)])

    return gather_k(codebook, indices)


def _st_body(zq_ref, z_ref, out_ref, acc_ref):
    i = pl.program_id(0)
    zq = zq_ref[...]
    zt = z_ref[...]
    diff = zq - zt
    out_ref[...] = zt + diff
    s = jnp.sum(diff * diff)

    @pl.when(i == 0)
    def _():
        acc_ref[0, 0] = s

    @pl.when(i > 0)
    def _():
        acc_ref[0, 0] = acc_ref[0, 0] + s


def _straight_through(z_q_flat, z_flat):
    return pl.pallas_call(
        _st_body,
        grid=(B_TOTAL // ST_TILE,),
        in_specs=[
            pl.BlockSpec((ST_TILE, D), lambda i: (i, 0)),
            pl.BlockSpec((ST_TILE, D), lambda i: (i, 0)),
        ],
        out_specs=(
            pl.BlockSpec((ST_TILE, D), lambda i: (i, 0)),
            pl.BlockSpec(memory_space=pltpu.SMEM),
        ),
        out_shape=(
            jax.ShapeDtypeStruct((B_TOTAL, D), jnp.float32),
            jax.ShapeDtypeStruct((1, 1), jnp.float32),
        ),
    )(z_q_flat, z_flat)


def kernel(z, embedding_weight, proj_weight):
    b, c, h, w = z.shape
    z_t = jnp.transpose(z, (0, 2, 3, 1))
    z_flat = z_t.reshape(-1, c)

    codebook, cnorm_col = _compute_codebook(embedding_weight, proj_weight)
    cnorm_row = cnorm_col.reshape(1, N_CB)

    indices = _compute_indices(z_flat, codebook, cnorm_row)
    z_q_flat = _gather_rows(codebook, indices)
    z_q_st, sq_sum = _straight_through(z_q_flat, z_flat)

    m = sq_sum[0, 0] / jnp.float32(B_TOTAL * D)
    commitment_loss = jnp.float32(0.25) * m
    codebook_loss = m
    loss = commitment_loss + codebook_loss

    z_q_out = jnp.transpose(z_q_st.reshape(b, h, w, c), (0, 3, 1, 2))
    indices_out = indices.reshape(b, h, w)
    return (z_q_out, loss, commitment_loss, codebook_loss, indices_out)


# speedup vs baseline: 1.5094x; 1.1118x over previous
import functools

import jax
import jax.numpy as jnp
from jax import lax
from jax.experimental import pallas as pl
from jax.experimental.pallas import tpu as pltpu
from jax.experimental.pallas import tpu_sc as plsc

B_TOTAL = 16384
D = 256
N_CB = 8192

M_TILE = 512
N_CHUNK = 2048

ST_TILE = 2048

_DIMNUMS_LAST = (((1,), (1,)), ((), ()))


def _codebook_body(e_ref, w_ref, cb_ref, cn_ref):
    cb = lax.dot_general(e_ref[...], w_ref[...], _DIMNUMS_LAST,
                         preferred_element_type=jnp.float32)
    cb_ref[...] = cb
    cn_ref[...] = jnp.sum(cb * cb, axis=1, keepdims=True)


def _compute_codebook(embedding_weight, proj_weight):
    return pl.pallas_call(
        _codebook_body,
        out_shape=(
            jax.ShapeDtypeStruct((N_CB, D), jnp.float32),
            jax.ShapeDtypeStruct((N_CB, 1), jnp.float32),
        ),
    )(embedding_weight, proj_weight)


def _argmin_body(z_ref, cb_ref, cn_ref, out_ref):
    z_t = z_ref[...]
    zn = jnp.sum(z_t * z_t, axis=1, keepdims=True)
    z_m2 = z_t * jnp.float32(-2.0)
    io_f = lax.broadcasted_iota(jnp.int32, (M_TILE, N_CHUNK), 1).astype(jnp.float32)
    run_min = jnp.full((M_TILE, 1), jnp.inf, jnp.float32)
    run_idx = jnp.zeros((M_TILE, 1), jnp.float32)
    for c in range(N_CB // N_CHUNK):
        cb_c = cb_ref[pl.ds(c * N_CHUNK, N_CHUNK), :]
        s2 = lax.dot_general(z_m2, cb_c, _DIMNUMS_LAST,
                             preferred_element_type=jnp.float32)
        d = (zn + cn_ref[:, pl.ds(c * N_CHUNK, N_CHUNK)]) + s2
        cmin = jnp.min(d, axis=1, keepdims=True)
        cidx = jnp.min(jnp.where(d == cmin, io_f, jnp.float32(N_CB)),
                       axis=1, keepdims=True) + jnp.float32(c * N_CHUNK)
        upd = cmin < run_min
        run_min = jnp.where(upd, cmin, run_min)
        run_idx = jnp.where(upd, cidx, run_idx)
    out_ref[...] = run_idx[:, 0].astype(jnp.int32)


def _compute_indices(z_flat, codebook, cnorm_row):
    return pl.pallas_call(
        _argmin_body,
        grid=(B_TOTAL // M_TILE,),
        in_specs=[
            pl.BlockSpec((M_TILE, D), lambda i: (i, 0)),
            pl.BlockSpec((N_CB, D), lambda i: (0, 0)),
            pl.BlockSpec((1, N_CB), lambda i: (0, 0)),
        ],
        out_specs=pl.BlockSpec((M_TILE,), lambda i: (i,)),
        out_shape=jax.ShapeDtypeStruct((B_TOTAL,), jnp.int32),
    )(z_flat, codebook, cnorm_row)


_NUM_SC_CORES = 2
_NUM_SC_SUBCORES = 16
_NW = _NUM_SC_CORES * _NUM_SC_SUBCORES
_ROWS_PER_W = B_TOTAL // _NW
_GCHUNK = 128


def _gather_rows(codebook, indices):
    mesh = plsc.VectorSubcoreMesh(core_axis_name="c", subcore_axis_name="s")

    @functools.partial(
        pl.kernel, mesh=mesh,
        out_type=jax.ShapeDtypeStruct((B_TOTAL, D), jnp.float32),
        scratch_types=[
            pltpu.VMEM((_GCHUNK,), jnp.int32),
            pltpu.VMEM((_GCHUNK, D), jnp.float32),
            pltpu.SemaphoreType.DMA,
        ],
    )
    def gather_k(table_hbm, idx_hbm, out_hbm, idx_v, rows_v, sem):
        wid = lax.axis_index("s") * _NUM_SC_CORES + lax.axis_index("c")
        base = wid * _ROWS_PER_W
        for c in range(_ROWS_PER_W // _GCHUNK):
            off = base + c * _GCHUNK
            pltpu.sync_copy(idx_hbm.at[pl.ds(off, _GCHUNK)], idx_v)
            pltpu.async_copy(table_hbm.at[idx_v], rows_v, sem).wait()
            pltpu.sync_copy(rows_v, out_hbm.at[pl.ds(off, _GCHUNK)])

    return gather_k(codebook, indices)


def _st_body(zq_ref, z_ref, out_ref, acc_ref):
    i = pl.program_id(0)
    zq = zq_ref[...]
    zt = z_ref[...]
    diff = zq - zt
    out_ref[...] = zt + diff
    s = jnp.sum(diff * diff)

    @pl.when(i == 0)
    def _():
        acc_ref[0, 0] = s

    @pl.when(i > 0)
    def _():
        acc_ref[0, 0] = acc_ref[0, 0] + s


def _straight_through(z_q_flat, z_flat):
    return pl.pallas_call(
        _st_body,
        grid=(B_TOTAL // ST_TILE,),
        in_specs=[
            pl.BlockSpec((ST_TILE, D), lambda i: (i, 0)),
            pl.BlockSpec((ST_TILE, D), lambda i: (i, 0)),
        ],
        out_specs=(
            pl.BlockSpec((ST_TILE, D), lambda i: (i, 0)),
            pl.BlockSpec(memory_space=pltpu.SMEM),
        ),
        out_shape=(
            jax.ShapeDtypeStruct((B_TOTAL, D), jnp.float32),
            jax.ShapeDtypeStruct((1, 1), jnp.float32),
        ),
    )(z_q_flat, z_flat)


def kernel(z, embedding_weight, proj_weight):
    b, c, h, w = z.shape
    z_t = jnp.transpose(z, (0, 2, 3, 1))
    z_flat = z_t.reshape(-1, c)

    codebook, cnorm_col = _compute_codebook(embedding_weight, proj_weight)
    cnorm_row = cnorm_col.reshape(1, N_CB)

    indices = _compute_indices(z_flat, codebook, cnorm_row)
    z_q_flat = _gather_rows(codebook, indices)
    z_q_st, sq_sum = _straight_through(z_q_flat, z_flat)

    m = sq_sum[0, 0] / jnp.float32(B_TOTAL * D)
    commitment_loss = jnp.float32(0.25) * m
    codebook_loss = m
    loss = commitment_loss + codebook_loss

    z_q_out = jnp.transpose(z_q_st.reshape(b, h, w, c), (0, 3, 1, 2))
    indices_out = indices.reshape(b, h, w)
    return (z_q_out, loss, commitment_loss, codebook_loss, indices_out)
